# Initial kernel scaffold; baseline (speedup 1.0000x reference)
import functools

import jax
import jax.numpy as jnp
from jax.experimental import pallas as pl
from jax.experimental.pallas import tpu as pltpu

BLK = 256
LEAK = 0.2


def _attention_factors(h, w_src, w_dst):
    es = jnp.dot(h, w_src, preferred_element_type=jnp.float32)
    ed = jnp.dot(h, w_dst, preferred_element_type=jnp.float32)
    ces = jnp.max(es, axis=0, keepdims=True)
    ced = jnp.max(ed, axis=0, keepdims=True)
    f1 = jnp.exp(es - ces)
    f2k = jnp.exp(LEAK * (es - ces) - (1.0 - LEAK) * (ces + ced))
    g1 = jnp.exp(ed - ced)
    g2 = jnp.exp(LEAK * (ed - ced))
    return es, f1, f2k, g1.T, g2.T, ed.T


def _block_attention(adj_blk, i, head, es, f1, f2k, g1t, g2t, edt):
    rows = pl.ds(i * BLK, BLK)
    es_b = es[rows, head : head + 1]
    z = es_b + edt[head : head + 1, :]
    w1 = f1[rows, head : head + 1] * g1t[head : head + 1, :]
    w2 = f2k[rows, head : head + 1] * g2t[head : head + 1, :]
    w = jnp.where(z > 0, w1, w2)
    p = w * adj_blk
    denom = jnp.sum(p, axis=1, keepdims=True)
    return p, denom


def _layer1_kernel(x_ref, adj_ref, w_ref, asrc_ref, adst_ref, out_ref,
                   h_scr, es_scr, f1_scr, f2k_scr, g1t_scr, g2t_scr, edt_scr,
                   *, heads, out1):
    i = pl.program_id(0)

    @pl.when(i == 0)
    def _precompute():
        h = jnp.dot(x_ref[...], w_ref[...], preferred_element_type=jnp.float32)
        h_scr[...] = h
        es, f1, f2k, g1t, g2t, edt = _attention_factors(
            h, asrc_ref[...], adst_ref[...])
        es_scr[...] = es
        f1_scr[...] = f1
        f2k_scr[...] = f2k
        g1t_scr[...] = g1t
        g2t_scr[...] = g2t
        edt_scr[...] = edt

    adj_blk = adj_ref[...]
    for head in range(heads):
        p, denom = _block_attention(adj_blk, i, head, es_scr[...], f1_scr[...],
                                    f2k_scr[...], g1t_scr[...], g2t_scr[...],
                                    edt_scr[...])
        num = jnp.dot(p, h_scr[:, head * out1:(head + 1) * out1],
                      preferred_element_type=jnp.float32)
        o = num / denom
        out_ref[:, head * out1:(head + 1) * out1] = jnp.where(
            o > 0, o, jnp.exp(o) - 1.0)


def _layer2_kernel(h1_ref, adj_ref, w_ref, asrc_ref, adst_ref, out_ref,
                   h_scr, es_scr, f1_scr, f2k_scr, g1t_scr, g2t_scr, edt_scr):
    i = pl.program_id(0)

    @pl.when(i == 0)
    def _precompute():
        h = jnp.dot(h1_ref[...], w_ref[...], preferred_element_type=jnp.float32)
        h_scr[...] = h
        es, f1, f2k, g1t, g2t, edt = _attention_factors(
            h, asrc_ref[...], adst_ref[...])
        es_scr[...] = es
        f1_scr[...] = f1
        f2k_scr[...] = f2k
        g1t_scr[...] = g1t
        g2t_scr[...] = g2t
        edt_scr[...] = edt

    adj_blk = adj_ref[...]
    p, denom = _block_attention(adj_blk, i, 0, es_scr[...], f1_scr[...],
                                f2k_scr[...], g1t_scr[...], g2t_scr[...],
                                edt_scr[...])
    num = jnp.dot(p, h_scr[...], preferred_element_type=jnp.float32)
    o = num / denom
    m = jnp.max(o, axis=1, keepdims=True)
    lse = jnp.log(jnp.sum(jnp.exp(o - m), axis=1, keepdims=True)) + m
    out_ref[...] = o - lse


def kernel(x, adj, W1, a1_src, a1_dst, W2, a2_src, a2_dst):
    n, ins = x.shape
    heads, _, out1 = W1.shape
    classes = W2.shape[2]
    hidden = heads * out1
    grid = (n // BLK,)

    w1f = jnp.transpose(W1, (1, 0, 2)).reshape(ins, hidden)
    eye = jnp.eye(heads, dtype=x.dtype)
    asrc1 = (a1_src[:, :, None] * eye[:, None, :]).reshape(hidden, heads)
    adst1 = (a1_dst[:, :, None] * eye[:, None, :]).reshape(hidden, heads)

    full = lambda r, c: pl.BlockSpec((r, c), lambda i: (0, 0))
    vec_scr = lambda r, c: pltpu.VMEM((r, c), jnp.float32)

    h1 = pl.pallas_call(
        functools.partial(_layer1_kernel, heads=heads, out1=out1),
        grid=grid,
        in_specs=[
            full(n, ins),
            pl.BlockSpec((BLK, n), lambda i: (i, 0)),
            full(ins, hidden),
            full(hidden, heads),
            full(hidden, heads),
        ],
        out_specs=pl.BlockSpec((BLK, hidden), lambda i: (i, 0)),
        out_shape=jax.ShapeDtypeStruct((n, hidden), jnp.float32),
        scratch_shapes=[
            vec_scr(n, hidden),
            vec_scr(n, heads),
            vec_scr(n, heads),
            vec_scr(n, heads),
            vec_scr(heads, n),
            vec_scr(heads, n),
            vec_scr(heads, n),
        ],
    )(x, adj, w1f, asrc1, adst1)

    h2 = pl.pallas_call(
        _layer2_kernel,
        grid=grid,
        in_specs=[
            full(n, hidden),
            pl.BlockSpec((BLK, n), lambda i: (i, 0)),
            full(hidden, classes),
            full(classes, 1),
            full(classes, 1),
        ],
        out_specs=pl.BlockSpec((BLK, classes), lambda i: (i, 0)),
        out_shape=jax.ShapeDtypeStruct((n, classes), jnp.float32),
        scratch_shapes=[
            vec_scr(n, classes),
            vec_scr(n, 1),
            vec_scr(n, 1),
            vec_scr(n, 1),
            vec_scr(1, n),
            vec_scr(1, n),
            vec_scr(1, n),
        ],
    )(h1, adj, W2[0], a2_src.reshape(classes, 1), a2_dst.reshape(classes, 1))

    return h2


# speedup vs baseline: 2.2048x; 2.2048x over previous
import functools

import jax
import jax.numpy as jnp
from jax.experimental import pallas as pl
from jax.experimental.pallas import tpu as pltpu

BLK = 256
LEAK = 0.2


def _attention_factors(h, w_src, w_dst):
    es = jnp.dot(h, w_src, preferred_element_type=jnp.float32)
    ed = jnp.dot(h, w_dst, preferred_element_type=jnp.float32)
    ces = jnp.max(es, axis=0, keepdims=True)
    ced = jnp.max(ed, axis=0, keepdims=True)
    f1 = jnp.exp(es - ces)
    f2k = jnp.exp(LEAK * (es - ces) - (1.0 - LEAK) * (ces + ced))
    g1 = jnp.exp(ed - ced)
    g2 = jnp.exp(LEAK * (ed - ced))
    return es, f1, f2k, g1.T, g2.T, ed.T


def _block_attention(adj_blk, i, head, es_ref, f1_ref, f2k_ref,
                     g1t_ref, g2t_ref, edt_ref):
    rows = pl.ds(i * BLK, BLK)
    z = es_ref[rows, head : head + 1] + edt_ref[head : head + 1, :]
    w1 = f1_ref[rows, head : head + 1] * g1t_ref[head : head + 1, :]
    w2 = f2k_ref[rows, head : head + 1] * g2t_ref[head : head + 1, :]
    w = jnp.where(z > 0, w1, w2)
    p = w * adj_blk
    denom = jnp.sum(p, axis=1, keepdims=True)
    return p, denom


def _layer1_kernel(x_ref, adj_ref, w_ref, asrc_ref, adst_ref, out_ref,
                   h_scr, es_scr, f1_scr, f2k_scr, g1t_scr, g2t_scr, edt_scr,
                   *, heads, out1):
    i = pl.program_id(0)

    @pl.when(i == 0)
    def _precompute():
        h = jnp.dot(x_ref[...], w_ref[...], preferred_element_type=jnp.float32)
        h_scr[...] = h
        es, f1, f2k, g1t, g2t, edt = _attention_factors(
            h, asrc_ref[...], adst_ref[...])
        es_scr[...] = es
        f1_scr[...] = f1
        f2k_scr[...] = f2k
        g1t_scr[...] = g1t
        g2t_scr[...] = g2t
        edt_scr[...] = edt

    adj_blk = adj_ref[...]
    for head in range(heads):
        p, denom = _block_attention(adj_blk, i, head, es_scr, f1_scr,
                                    f2k_scr, g1t_scr, g2t_scr, edt_scr)
        num = jnp.dot(p, h_scr[:, head * out1:(head + 1) * out1],
                      preferred_element_type=jnp.float32)
        o = num / denom
        out_ref[:, head * out1:(head + 1) * out1] = jnp.where(
            o > 0, o, jnp.exp(o) - 1.0)


def _layer2_kernel(h1_ref, adj_ref, w_ref, asrc_ref, adst_ref, out_ref,
                   h_scr, es_scr, f1_scr, f2k_scr, g1t_scr, g2t_scr, edt_scr):
    i = pl.program_id(0)

    @pl.when(i == 0)
    def _precompute():
        h = jnp.dot(h1_ref[...], w_ref[...], preferred_element_type=jnp.float32)
        h_scr[...] = h
        es, f1, f2k, g1t, g2t, edt = _attention_factors(
            h, asrc_ref[...], adst_ref[...])
        es_scr[...] = es
        f1_scr[...] = f1
        f2k_scr[...] = f2k
        g1t_scr[...] = g1t
        g2t_scr[...] = g2t
        edt_scr[...] = edt

    adj_blk = adj_ref[...]
    p, denom = _block_attention(adj_blk, i, 0, es_scr, f1_scr,
                                f2k_scr, g1t_scr, g2t_scr, edt_scr)
    num = jnp.dot(p, h_scr[...], preferred_element_type=jnp.float32)
    o = num / denom
    m = jnp.max(o, axis=1, keepdims=True)
    lse = jnp.log(jnp.sum(jnp.exp(o - m), axis=1, keepdims=True)) + m
    out_ref[...] = o - lse


def kernel(x, adj, W1, a1_src, a1_dst, W2, a2_src, a2_dst):
    n, ins = x.shape
    heads, _, out1 = W1.shape
    classes = W2.shape[2]
    hidden = heads * out1
    grid = (n // BLK,)

    w1f = jnp.transpose(W1, (1, 0, 2)).reshape(ins, hidden)
    eye = jnp.eye(heads, dtype=x.dtype)
    asrc1 = (a1_src[:, :, None] * eye[:, None, :]).reshape(hidden, heads)
    adst1 = (a1_dst[:, :, None] * eye[:, None, :]).reshape(hidden, heads)

    full = lambda r, c: pl.BlockSpec((r, c), lambda i: (0, 0))
    vec_scr = lambda r, c: pltpu.VMEM((r, c), jnp.float32)

    h1 = pl.pallas_call(
        functools.partial(_layer1_kernel, heads=heads, out1=out1),
        grid=grid,
        in_specs=[
            full(n, ins),
            pl.BlockSpec((BLK, n), lambda i: (i, 0)),
            full(ins, hidden),
            full(hidden, heads),
            full(hidden, heads),
        ],
        out_specs=pl.BlockSpec((BLK, hidden), lambda i: (i, 0)),
        out_shape=jax.ShapeDtypeStruct((n, hidden), jnp.float32),
        scratch_shapes=[
            vec_scr(n, hidden),
            vec_scr(n, heads),
            vec_scr(n, heads),
            vec_scr(n, heads),
            vec_scr(heads, n),
            vec_scr(heads, n),
            vec_scr(heads, n),
        ],
    )(x, adj, w1f, asrc1, adst1)

    h2 = pl.pallas_call(
        _layer2_kernel,
        grid=grid,
        in_specs=[
            full(n, hidden),
            pl.BlockSpec((BLK, n), lambda i: (i, 0)),
            full(hidden, classes),
            full(classes, 1),
            full(classes, 1),
        ],
        out_specs=pl.BlockSpec((BLK, classes), lambda i: (i, 0)),
        out_shape=jax.ShapeDtypeStruct((n, classes), jnp.float32),
        scratch_shapes=[
            vec_scr(n, classes),
            vec_scr(n, 1),
            vec_scr(n, 1),
            vec_scr(n, 1),
            vec_scr(1, n),
            vec_scr(1, n),
            vec_scr(1, n),
        ],
    )(h1, adj, W2[0], a2_src.reshape(classes, 1), a2_dst.reshape(classes, 1))

    return h2


# speedup vs baseline: 2.8258x; 1.2817x over previous
import functools

import jax
import jax.numpy as jnp
from jax.experimental import pallas as pl
from jax.experimental.pallas import tpu as pltpu

BLK = 256
LEAK = 0.2


def _attention_factors(h, w_src, w_dst, dtype):
    es = jnp.dot(h, w_src, preferred_element_type=jnp.float32)
    ed = jnp.dot(h, w_dst, preferred_element_type=jnp.float32)
    ces = jnp.max(es, axis=0, keepdims=True)
    ced = jnp.max(ed, axis=0, keepdims=True)
    f1 = jnp.exp(es - ces)
    f2k = jnp.exp(LEAK * (es - ces) - (1.0 - LEAK) * (ces + ced))
    g1 = jnp.exp(ed - ced)
    g2 = jnp.exp(LEAK * (ed - ced))
    c = lambda v: v.astype(dtype)
    return c(es), c(f1), c(f2k), c(g1.T), c(g2.T), c(ed.T)


def _block_weights(adj_blk, i, head, es_ref, f1_ref, f2k_ref,
                   g1t_ref, g2t_ref, edt_ref):
    rows = pl.ds(i * BLK, BLK)
    z = es_ref[rows, head : head + 1] + edt_ref[head : head + 1, :]
    w1 = f1_ref[rows, head : head + 1] * g1t_ref[head : head + 1, :]
    w2 = f2k_ref[rows, head : head + 1] * g2t_ref[head : head + 1, :]
    w = jnp.where(z > 0, w1, w2)
    return w * adj_blk


def _layer1_kernel(x_ref, adj_ref, w_ref, asrc_ref, adst_ref, out_ref,
                   hext_scr, es_scr, f1_scr, f2k_scr, g1t_scr, g2t_scr,
                   edt_scr, *, heads, out1):
    i = pl.program_id(0)
    ext = out1 + 1

    @pl.when(i == 0)
    def _precompute():
        h = jnp.dot(x_ref[...], w_ref[...], preferred_element_type=jnp.float32)
        es, f1, f2k, g1t, g2t, edt = _attention_factors(
            h, asrc_ref[...], adst_ref[...], jnp.bfloat16)
        es_scr[...] = es
        f1_scr[...] = f1
        f2k_scr[...] = f2k
        g1t_scr[...] = g1t
        g2t_scr[...] = g2t
        edt_scr[...] = edt
        h16 = h.astype(jnp.bfloat16)
        for head in range(heads):
            hext_scr[:, head * ext:head * ext + out1] = (
                h16[:, head * out1:(head + 1) * out1])
            hext_scr[:, head * ext + out1:(head + 1) * ext] = jnp.ones(
                (h.shape[0], 1), jnp.bfloat16)

    adj_blk = adj_ref[...]
    for head in range(heads):
        p = _block_weights(adj_blk, i, head, es_scr, f1_scr, f2k_scr,
                           g1t_scr, g2t_scr, edt_scr)
        ne = jnp.dot(p, hext_scr[:, head * ext:(head + 1) * ext],
                     preferred_element_type=jnp.float32)
        o = ne[:, :out1] / ne[:, out1:]
        out_ref[:, head * out1:(head + 1) * out1] = jnp.where(
            o > 0, o, jnp.exp(o) - 1.0)


def _layer2_kernel(h1_ref, adj_ref, w_ref, asrc_ref, adst_ref, out_ref,
                   h_scr, es_scr, f1_scr, f2k_scr, g1t_scr, g2t_scr, edt_scr):
    i = pl.program_id(0)

    @pl.when(i == 0)
    def _precompute():
        h = jnp.dot(h1_ref[...], w_ref[...], preferred_element_type=jnp.float32)
        h_scr[...] = h
        es, f1, f2k, g1t, g2t, edt = _attention_factors(
            h, asrc_ref[...], adst_ref[...], jnp.float32)
        es_scr[...] = es
        f1_scr[...] = f1
        f2k_scr[...] = f2k
        g1t_scr[...] = g1t
        g2t_scr[...] = g2t
        edt_scr[...] = edt

    p = _block_weights(adj_ref[...], i, 0, es_scr, f1_scr, f2k_scr,
                       g1t_scr, g2t_scr, edt_scr)
    denom = jnp.sum(p, axis=1, keepdims=True)
    num = jnp.dot(p, h_scr[...], preferred_element_type=jnp.float32)
    o = num / denom
    m = jnp.max(o, axis=1, keepdims=True)
    lse = jnp.log(jnp.sum(jnp.exp(o - m), axis=1, keepdims=True)) + m
    out_ref[...] = o - lse


def kernel(x, adj, W1, a1_src, a1_dst, W2, a2_src, a2_dst):
    n, ins = x.shape
    heads, _, out1 = W1.shape
    classes = W2.shape[2]
    hidden = heads * out1
    grid = (n // BLK,)

    adj16 = adj.astype(jnp.bfloat16)

    w1f = jnp.transpose(W1, (1, 0, 2)).reshape(ins, hidden)
    eye = jnp.eye(heads, dtype=x.dtype)
    asrc1 = (a1_src[:, :, None] * eye[:, None, :]).reshape(hidden, heads)
    adst1 = (a1_dst[:, :, None] * eye[:, None, :]).reshape(hidden, heads)

    full = lambda r, c: pl.BlockSpec((r, c), lambda i: (0, 0))
    f32_scr = lambda r, c: pltpu.VMEM((r, c), jnp.float32)
    bf16_scr = lambda r, c: pltpu.VMEM((r, c), jnp.bfloat16)

    h1 = pl.pallas_call(
        functools.partial(_layer1_kernel, heads=heads, out1=out1),
        grid=grid,
        in_specs=[
            full(n, ins),
            pl.BlockSpec((BLK, n), lambda i: (i, 0)),
            full(ins, hidden),
            full(hidden, heads),
            full(hidden, heads),
        ],
        out_specs=pl.BlockSpec((BLK, hidden), lambda i: (i, 0)),
        out_shape=jax.ShapeDtypeStruct((n, hidden), jnp.float32),
        scratch_shapes=[
            bf16_scr(n, (out1 + 1) * heads),
            bf16_scr(n, heads),
            bf16_scr(n, heads),
            bf16_scr(n, heads),
            bf16_scr(heads, n),
            bf16_scr(heads, n),
            bf16_scr(heads, n),
        ],
    )(x, adj16, w1f, asrc1, adst1)

    h2 = pl.pallas_call(
        _layer2_kernel,
        grid=grid,
        in_specs=[
            full(n, hidden),
            pl.BlockSpec((BLK, n), lambda i: (i, 0)),
            full(hidden, classes),
            full(classes, 1),
            full(classes, 1),
        ],
        out_specs=pl.BlockSpec((BLK, classes), lambda i: (i, 0)),
        out_shape=jax.ShapeDtypeStruct((n, classes), jnp.float32),
        scratch_shapes=[
            f32_scr(n, classes),
            f32_scr(n, 1),
            f32_scr(n, 1),
            f32_scr(n, 1),
            f32_scr(1, n),
            f32_scr(1, n),
            f32_scr(1, n),
        ],
    )(h1, adj, W2[0], a2_src.reshape(classes, 1), a2_dst.reshape(classes, 1))

    return h2


# speedup vs baseline: 3.1528x; 1.1157x over previous
import functools

import jax
import jax.numpy as jnp
from jax.experimental import pallas as pl
from jax.experimental.pallas import tpu as pltpu

BLK = 256
LEAK = 0.2


def _attention_factors(h, w_src, w_dst, dtype):
    es = jnp.dot(h, w_src, preferred_element_type=jnp.float32)
    ed = jnp.dot(h, w_dst, preferred_element_type=jnp.float32)
    ces = jnp.max(es, axis=0, keepdims=True)
    ced = jnp.max(ed, axis=0, keepdims=True)
    f1 = jnp.exp(es - ces)
    f2k = jnp.exp(LEAK * (es - ces) - (1.0 - LEAK) * (ces + ced))
    g1 = jnp.exp(ed - ced)
    g2 = jnp.exp(LEAK * (ed - ced))
    c = lambda v: v.astype(dtype)
    return c(f1), c(f2k), c(g1.T), c(g2.T)


def _block_weights(adj_blk, i, head, f1_ref, f2k_ref, g1t_ref, g2t_ref):
    rows = pl.ds(i * BLK, BLK)
    w1 = f1_ref[rows, head : head + 1] * g1t_ref[head : head + 1, :]
    w2 = f2k_ref[rows, head : head + 1] * g2t_ref[head : head + 1, :]
    return jnp.maximum(w1, w2) * adj_blk


def _layer1_kernel(x_ref, adj_ref, w_ref, asrc_ref, adst_ref, out_ref,
                   hext_scr, f1_scr, f2k_scr, g1t_scr, g2t_scr,
                   *, heads, out1):
    i = pl.program_id(0)
    ext = out1 + 1

    @pl.when(i == 0)
    def _precompute():
        h = jnp.dot(x_ref[...], w_ref[...], preferred_element_type=jnp.float32)
        f1, f2k, g1t, g2t = _attention_factors(
            h, asrc_ref[...], adst_ref[...], jnp.bfloat16)
        f1_scr[...] = f1
        f2k_scr[...] = f2k
        g1t_scr[...] = g1t
        g2t_scr[...] = g2t
        h16 = h.astype(jnp.bfloat16)
        for head in range(heads):
            hext_scr[:, head * ext:head * ext + out1] = (
                h16[:, head * out1:(head + 1) * out1])
            hext_scr[:, head * ext + out1:(head + 1) * ext] = jnp.ones(
                (h.shape[0], 1), jnp.bfloat16)

    adj_blk = adj_ref[...]
    for head in range(heads):
        p = _block_weights(adj_blk, i, head, f1_scr, f2k_scr,
                           g1t_scr, g2t_scr)
        ne = jnp.dot(p, hext_scr[:, head * ext:(head + 1) * ext],
                     preferred_element_type=jnp.float32)
        o = ne[:, :out1] / ne[:, out1:]
        out_ref[:, head * out1:(head + 1) * out1] = jnp.where(
            o > 0, o, jnp.exp(o) - 1.0)


def _layer2_kernel(h1_ref, adj_ref, w_ref, asrc_ref, adst_ref, out_ref,
                   hext_scr, f1_scr, f2k_scr, g1t_scr, g2t_scr):
    i = pl.program_id(0)
    classes = hext_scr.shape[1] - 1

    @pl.when(i == 0)
    def _precompute():
        h = jnp.dot(h1_ref[...], w_ref[...], preferred_element_type=jnp.float32)
        f1, f2k, g1t, g2t = _attention_factors(
            h, asrc_ref[...], adst_ref[...], jnp.bfloat16)
        f1_scr[...] = f1
        f2k_scr[...] = f2k
        g1t_scr[...] = g1t
        g2t_scr[...] = g2t
        hext_scr[:, :classes] = h.astype(jnp.bfloat16)
        hext_scr[:, classes:] = jnp.ones((h.shape[0], 1), jnp.bfloat16)

    p = _block_weights(adj_ref[...], i, 0, f1_scr, f2k_scr,
                       g1t_scr, g2t_scr)
    ne = jnp.dot(p, hext_scr[...], preferred_element_type=jnp.float32)
    o = ne[:, :classes] / ne[:, classes:]
    m = jnp.max(o, axis=1, keepdims=True)
    lse = jnp.log(jnp.sum(jnp.exp(o - m), axis=1, keepdims=True)) + m
    out_ref[...] = o - lse


def kernel(x, adj, W1, a1_src, a1_dst, W2, a2_src, a2_dst):
    n, ins = x.shape
    heads, _, out1 = W1.shape
    classes = W2.shape[2]
    hidden = heads * out1
    grid = (n // BLK,)

    adj16 = adj.astype(jnp.bfloat16)

    w1f = jnp.transpose(W1, (1, 0, 2)).reshape(ins, hidden)
    eye = jnp.eye(heads, dtype=x.dtype)
    asrc1 = (a1_src[:, :, None] * eye[:, None, :]).reshape(hidden, heads)
    adst1 = (a1_dst[:, :, None] * eye[:, None, :]).reshape(hidden, heads)

    full = lambda r, c: pl.BlockSpec((r, c), lambda i: (0, 0))
    f32_scr = lambda r, c: pltpu.VMEM((r, c), jnp.float32)
    bf16_scr = lambda r, c: pltpu.VMEM((r, c), jnp.bfloat16)

    h1 = pl.pallas_call(
        functools.partial(_layer1_kernel, heads=heads, out1=out1),
        grid=grid,
        in_specs=[
            full(n, ins),
            pl.BlockSpec((BLK, n), lambda i: (i, 0)),
            full(ins, hidden),
            full(hidden, heads),
            full(hidden, heads),
        ],
        out_specs=pl.BlockSpec((BLK, hidden), lambda i: (i, 0)),
        out_shape=jax.ShapeDtypeStruct((n, hidden), jnp.float32),
        scratch_shapes=[
            bf16_scr(n, (out1 + 1) * heads),
            bf16_scr(n, heads),
            bf16_scr(n, heads),
            bf16_scr(heads, n),
            bf16_scr(heads, n),
        ],
    )(x, adj16, w1f, asrc1, adst1)

    h2 = pl.pallas_call(
        _layer2_kernel,
        grid=grid,
        in_specs=[
            full(n, hidden),
            pl.BlockSpec((BLK, n), lambda i: (i, 0)),
            full(hidden, classes),
            full(classes, 1),
            full(classes, 1),
        ],
        out_specs=pl.BlockSpec((BLK, classes), lambda i: (i, 0)),
        out_shape=jax.ShapeDtypeStruct((n, classes), jnp.float32),
        scratch_shapes=[
            bf16_scr(n, classes + 1),
            bf16_scr(n, 1),
            bf16_scr(n, 1),
            bf16_scr(1, n),
            bf16_scr(1, n),
        ],
    )(h1, adj16, W2[0], a2_src.reshape(classes, 1), a2_dst.reshape(classes, 1))

    return h2


# speedup vs baseline: 3.4106x; 1.0818x over previous
import functools

import jax
import jax.numpy as jnp
from jax.experimental import pallas as pl
from jax.experimental.pallas import tpu as pltpu

BLK = 512
LEAK = 0.2


def _attention_factors(h, w_src, w_dst, dtype):
    es = jnp.dot(h, w_src, preferred_element_type=jnp.float32)
    ed = jnp.dot(h, w_dst, preferred_element_type=jnp.float32)
    ces = jnp.max(es, axis=0, keepdims=True)
    ced = jnp.max(ed, axis=0, keepdims=True)
    f1 = jnp.exp(es - ces)
    f2k = jnp.exp(LEAK * (es - ces) - (1.0 - LEAK) * (ces + ced))
    g1 = jnp.exp(ed - ced)
    g2 = jnp.exp(LEAK * (ed - ced))
    c = lambda v: v.astype(dtype)
    return c(f1), c(f2k), c(g1.T), c(g2.T)


def _block_weights(adj_blk, i, head, f1_ref, f2k_ref, g1t_ref, g2t_ref):
    rows = pl.ds(i * BLK, BLK)
    w1 = f1_ref[rows, head : head + 1] * g1t_ref[head : head + 1, :]
    w2 = f2k_ref[rows, head : head + 1] * g2t_ref[head : head + 1, :]
    return jnp.maximum(w1, w2) * adj_blk


def _layer1_kernel(x_ref, adj_ref, w_ref, asrc_ref, adst_ref, out_ref,
                   hext_scr, f1_scr, f2k_scr, g1t_scr, g2t_scr,
                   *, heads, out1):
    i = pl.program_id(0)
    ext = out1 + 1

    @pl.when(i == 0)
    def _precompute():
        h = jnp.dot(x_ref[...], w_ref[...], preferred_element_type=jnp.float32)
        f1, f2k, g1t, g2t = _attention_factors(
            h, asrc_ref[...], adst_ref[...], jnp.bfloat16)
        f1_scr[...] = f1
        f2k_scr[...] = f2k
        g1t_scr[...] = g1t
        g2t_scr[...] = g2t
        h16 = h.astype(jnp.bfloat16)
        for head in range(heads):
            hext_scr[:, head * ext:head * ext + out1] = (
                h16[:, head * out1:(head + 1) * out1])
            hext_scr[:, head * ext + out1:(head + 1) * ext] = jnp.ones(
                (h.shape[0], 1), jnp.bfloat16)

    adj_blk = adj_ref[...]
    for head in range(heads):
        p = _block_weights(adj_blk, i, head, f1_scr, f2k_scr,
                           g1t_scr, g2t_scr)
        ne = jnp.dot(p, hext_scr[:, head * ext:(head + 1) * ext],
                     preferred_element_type=jnp.float32)
        o = ne[:, :out1] / ne[:, out1:]
        out_ref[:, head * out1:(head + 1) * out1] = jnp.where(
            o > 0, o, jnp.exp(o) - 1.0)


def _layer2_kernel(h1_ref, adj_ref, w_ref, asrc_ref, adst_ref, out_ref,
                   hext_scr, f1_scr, f2k_scr, g1t_scr, g2t_scr):
    i = pl.program_id(0)
    classes = hext_scr.shape[1] - 1

    @pl.when(i == 0)
    def _precompute():
        h = jnp.dot(h1_ref[...], w_ref[...], preferred_element_type=jnp.float32)
        f1, f2k, g1t, g2t = _attention_factors(
            h, asrc_ref[...], adst_ref[...], jnp.bfloat16)
        f1_scr[...] = f1
        f2k_scr[...] = f2k
        g1t_scr[...] = g1t
        g2t_scr[...] = g2t
        hext_scr[:, :classes] = h.astype(jnp.bfloat16)
        hext_scr[:, classes:] = jnp.ones((h.shape[0], 1), jnp.bfloat16)

    p = _block_weights(adj_ref[...], i, 0, f1_scr, f2k_scr,
                       g1t_scr, g2t_scr)
    ne = jnp.dot(p, hext_scr[...], preferred_element_type=jnp.float32)
    o = ne[:, :classes] / ne[:, classes:]
    m = jnp.max(o, axis=1, keepdims=True)
    lse = jnp.log(jnp.sum(jnp.exp(o - m), axis=1, keepdims=True)) + m
    out_ref[...] = o - lse


def kernel(x, adj, W1, a1_src, a1_dst, W2, a2_src, a2_dst):
    n, ins = x.shape
    heads, _, out1 = W1.shape
    classes = W2.shape[2]
    hidden = heads * out1
    grid = (n // BLK,)

    adj16 = adj.astype(jnp.bfloat16)

    w1f = jnp.transpose(W1, (1, 0, 2)).reshape(ins, hidden)
    eye = jnp.eye(heads, dtype=x.dtype)
    asrc1 = (a1_src[:, :, None] * eye[:, None, :]).reshape(hidden, heads)
    adst1 = (a1_dst[:, :, None] * eye[:, None, :]).reshape(hidden, heads)

    full = lambda r, c: pl.BlockSpec((r, c), lambda i: (0, 0))
    f32_scr = lambda r, c: pltpu.VMEM((r, c), jnp.float32)
    bf16_scr = lambda r, c: pltpu.VMEM((r, c), jnp.bfloat16)

    h1 = pl.pallas_call(
        functools.partial(_layer1_kernel, heads=heads, out1=out1),
        grid=grid,
        in_specs=[
            full(n, ins),
            pl.BlockSpec((BLK, n), lambda i: (i, 0)),
            full(ins, hidden),
            full(hidden, heads),
            full(hidden, heads),
        ],
        out_specs=pl.BlockSpec((BLK, hidden), lambda i: (i, 0)),
        out_shape=jax.ShapeDtypeStruct((n, hidden), jnp.float32),
        scratch_shapes=[
            bf16_scr(n, (out1 + 1) * heads),
            bf16_scr(n, heads),
            bf16_scr(n, heads),
            bf16_scr(heads, n),
            bf16_scr(heads, n),
        ],
    )(x, adj16, w1f, asrc1, adst1)

    h2 = pl.pallas_call(
        _layer2_kernel,
        grid=grid,
        in_specs=[
            full(n, hidden),
            pl.BlockSpec((BLK, n), lambda i: (i, 0)),
            full(hidden, classes),
            full(classes, 1),
            full(classes, 1),
        ],
        out_specs=pl.BlockSpec((BLK, classes), lambda i: (i, 0)),
        out_shape=jax.ShapeDtypeStruct((n, classes), jnp.float32),
        scratch_shapes=[
            bf16_scr(n, classes + 1),
            bf16_scr(n, 1),
            bf16_scr(n, 1),
            bf16_scr(1, n),
            bf16_scr(1, n),
        ],
    )(h1, adj16, W2[0], a2_src.reshape(classes, 1), a2_dst.reshape(classes, 1))

    return h2
